# Initial kernel scaffold; baseline (speedup 1.0000x reference)
import jax
import jax.numpy as jnp
from jax.experimental import pallas as pl


def kernel(x, emb_W, lin_W, lin_bias, W1, b1, g1, be1, W2, b2, g2, be2, W3, b3):
    raise NotImplementedError("write your pallas kernel here")


# speedup vs baseline: 7.2991x; 7.2991x over previous
import functools

import jax
import jax.numpy as jnp
from jax import lax
from jax.experimental import pallas as pl
from jax.experimental.pallas import tpu as pltpu
from jax.experimental.pallas import tpu_sc as plsc

NUM_FIELDS = 26
FIELD_DIM = 100000
EMBED_DIM = 16
BATCH = 16384
BF = BATCH * NUM_FIELDS
BN_EPS = 1e-5

NC, NS = 2, 16
NW = NC * NS
PER_W = BF // NW
CH = 1664
NCH = PER_W // CH
SB = BATCH // NW

MLP_BLK = 1024


def _sc_body(idx_emb_hbm, idx_lin_hbm, emb_hbm, lin_hbm, h_out, lin_out,
             idx_c, rows_v, idxl_v, vals_v, linb_v, sem_e, sem_l):
    w = lax.axis_index("s") * NC + lax.axis_index("c")
    base = w * PER_W

    pltpu.sync_copy(idx_lin_hbm.at[pl.ds(base, PER_W)], idxl_v)
    lin_cp = pltpu.async_copy(lin_hbm.at[idxl_v], vals_v, sem_l)

    for c in range(NCH):
        pltpu.sync_copy(idx_emb_hbm.at[pl.ds(base + c * CH, CH)], idx_c)
        pltpu.async_copy(emb_hbm.at[idx_c], rows_v, sem_e).wait()
        pltpu.sync_copy(rows_v, h_out.at[pl.ds(base + c * CH, CH)])

    lin_cp.wait()

    def gbody(g, _):
        col = g * 16
        acc = vals_v[pl.ds(col, 16)]
        for f in range(1, NUM_FIELDS):
            acc = acc + vals_v[pl.ds(f * SB + col, 16)]
        linb_v[pl.ds(col, 16)] = acc
        return 0

    lax.fori_loop(0, SB // 16, gbody, 0)
    pltpu.sync_copy(linb_v, lin_out.at[pl.ds(w * SB, SB)])


_sc_gather = functools.partial(
    pl.kernel,
    out_type=[
        jax.ShapeDtypeStruct((BF, EMBED_DIM), jnp.float32),
        jax.ShapeDtypeStruct((BATCH,), jnp.float32),
    ],
    mesh=plsc.VectorSubcoreMesh(core_axis_name="c", subcore_axis_name="s"),
    scratch_types=[
        pltpu.VMEM((CH,), jnp.int32),
        pltpu.VMEM((CH, EMBED_DIM), jnp.float32),
        pltpu.VMEM((PER_W,), jnp.int32),
        pltpu.VMEM((PER_W,), jnp.float32),
        pltpu.VMEM((SB,), jnp.float32),
        pltpu.SemaphoreType.DMA,
        pltpu.SemaphoreType.DMA,
    ],
    compiler_params=pltpu.CompilerParams(use_tc_tiling_on_sc=False),
)(_sc_body)


def _mlp_body(h_ref, lin_ref, W1_ref, b1_ref, g1_ref, be1_ref,
              W2_ref, b2_ref, g2_ref, be2_ref, W3_ref, b3_ref, lb_ref,
              out_ref):
    inv = 1.0 / (1.0 + BN_EPS) ** 0.5
    s1 = g1_ref[...] * inv
    t1 = b1_ref[...] * s1 + be1_ref[...]
    a = jnp.dot(h_ref[...], W1_ref[...], preferred_element_type=jnp.float32)
    a = jnp.maximum(a * s1 + t1, 0.0)
    s2 = g2_ref[...] * inv
    t2 = b2_ref[...] * s2 + be2_ref[...]
    a = jnp.dot(a, W2_ref[...], preferred_element_type=jnp.float32)
    a = jnp.maximum(a * s2 + t2, 0.0)
    deep = jnp.sum(a * W3_ref[...], axis=1, keepdims=True)
    out_ref[...] = deep + b3_ref[...] + lb_ref[...] + lin_ref[...]


def _mlp(h2d, lin2d, W1, b1, g1, be1, W2, b2, g2, be2, W3r, b3, lbias):
    full = lambda shape: pl.BlockSpec(shape, lambda i: (0, 0))
    return pl.pallas_call(
        _mlp_body,
        grid=(BATCH // MLP_BLK,),
        in_specs=[
            pl.BlockSpec((MLP_BLK, NUM_FIELDS * EMBED_DIM), lambda i: (i, 0)),
            pl.BlockSpec((MLP_BLK, 1), lambda i: (i, 0)),
            full((NUM_FIELDS * EMBED_DIM, 256)),
            full((1, 256)), full((1, 256)), full((1, 256)),
            full((256, 128)),
            full((1, 128)), full((1, 128)), full((1, 128)),
            full((1, 128)),
            full((1, 1)), full((1, 1)),
        ],
        out_specs=pl.BlockSpec((MLP_BLK, 1), lambda i: (i, 0)),
        out_shape=jax.ShapeDtypeStruct((BATCH, 1), jnp.float32),
    )(h2d, lin2d, W1, b1, g1, be1, W2, b2, g2, be2, W3r, b3, lbias)


def kernel(x, emb_W, lin_W, lin_bias, W1, b1, g1, be1, W2, b2, g2, be2,
           W3, b3):
    offs = jnp.arange(NUM_FIELDS, dtype=jnp.int32) * FIELD_DIM
    idx = x.astype(jnp.int32) + offs[None, :]
    idx_emb = idx.reshape(-1)
    idx_lin = (idx.T.reshape(NUM_FIELDS, NW, SB)
               .transpose(1, 0, 2).reshape(-1))
    h, lin_sum = _sc_gather(idx_emb, idx_lin, emb_W, lin_W.reshape(-1))
    out = _mlp(
        h.reshape(BATCH, NUM_FIELDS * EMBED_DIM),
        lin_sum.reshape(BATCH, 1),
        W1, b1.reshape(1, -1), g1.reshape(1, -1), be1.reshape(1, -1),
        W2, b2.reshape(1, -1), g2.reshape(1, -1), be2.reshape(1, -1),
        W3.reshape(1, -1), b3.reshape(1, 1), lin_bias.reshape(1, 1),
    )
    return jnp.squeeze(out, axis=1)
